# baseline (speedup 1.0000x reference)
import jax
import jax.numpy as jnp
from jax import lax
from jax.experimental import pallas as pl
from jax.experimental.pallas import tpu as pltpu
from jax.experimental.pallas import tpu_sc as plsc

N = 10000
E = 320000
IN_CH = 128
HID = 64
G = 64
TASKS = 2

NC, NS = 2, 16
NW = NC * NS
CHUNK = 128
NBUF = 6
DEPTH = 3
CPT0 = 36
CPT1 = 126
CPT = max(CPT0, CPT1)
E_PAD = NS * CHUNK * (CPT0 + CPT1)
NPAD = 10240
TRASH = N
RPS = NPAD // NS
DW = 16

_P = jax.lax.Precision.HIGHEST
_mesh = plsc.VectorSubcoreMesh(core_axis_name="c", subcore_axis_name="s")
_SC_PARAMS = pltpu.CompilerParams(use_tc_tiling_on_sc=False)


def _deg_kernel(sd_hbm, ones_hbm, z_hbm, out_hbm, idx_all, ones_v, dacc, sem):
    cid = lax.axis_index("c")
    sid = lax.axis_index("s")
    wid = cid * NS + sid
    nj = jnp.where(cid == 0, CPT0, CPT1)
    pltpu.sync_copy(z_hbm, dacc.at[pl.ds(sid * RPS, RPS)])
    pltpu.sync_copy(ones_hbm, ones_v)
    pltpu.sync_copy(sd_hbm.at[wid], idx_all)
    plsc.subcore_barrier()

    def s_desc(j, b):
        return pltpu.make_async_copy(
            ones_v, dacc.at[idx_all.at[j, 1]], sem.at[b])

    @pl.loop(0, CPT)
    def _(j):
        @pl.when(j < nj)
        def _():
            b = lax.rem(j, NBUF)

            @pl.when(j >= NBUF)
            def _():
                s_desc(j - NBUF, b).wait()

            s_desc(j, b).start(add=True)

    for b in range(NBUF):
        s_desc(nj - NBUF + b, b).wait()
    plsc.subcore_barrier()
    pltpu.sync_copy(dacc.at[pl.ds(sid * RPS, RPS)],
                    out_hbm.at[cid, pl.ds(sid * RPS, RPS)])


def _deg_call(sd, ones_blk, zeros_blk):
    return pl.kernel(
        _deg_kernel,
        out_type=jax.ShapeDtypeStruct((NC, NPAD, DW), jnp.float32),
        mesh=_mesh,
        compiler_params=_SC_PARAMS,
        scratch_types=[
            pltpu.VMEM((CPT, 2, CHUNK), jnp.int32),
            pltpu.VMEM((CHUNK, DW), jnp.float32),
            pltpu.VMEM_SHARED((NPAD, DW), jnp.float32),
            pltpu.SemaphoreType.DMA((NBUF,)),
        ],
    )(sd, ones_blk, zeros_blk)


def _agg_kernel(hp_hbm, sd_hbm, z_hbm, out_hbm, idx_all, rows, acc,
                semg, sems):
    cid = lax.axis_index("c")
    sid = lax.axis_index("s")
    wid = cid * NS + sid
    nj = jnp.where(cid == 0, CPT0, CPT1)
    sl = pl.ds(sid * RPS, RPS)
    pltpu.sync_copy(z_hbm, acc.at[sl])
    pltpu.sync_copy(sd_hbm.at[wid], idx_all)
    plsc.subcore_barrier()

    def g_desc(j, b):
        return pltpu.make_async_copy(
            hp_hbm.at[idx_all.at[j, 0]], rows.at[b], semg.at[b])

    def s_desc(j, b):
        return pltpu.make_async_copy(
            rows.at[b], acc.at[idx_all.at[j, 1]], sems.at[b])

    @pl.loop(0, CPT + DEPTH)
    def _(j):
        @pl.when(j < nj)
        def _():
            b = lax.rem(j, NBUF)

            @pl.when(j >= NBUF)
            def _():
                s_desc(j - NBUF, b).wait()

            g_desc(j, b).start()

        @pl.when(jnp.logical_and(j >= DEPTH, j < nj + DEPTH))
        def _():
            jd = j - DEPTH
            bd = lax.rem(jd, NBUF)
            g_desc(jd, bd).wait()
            s_desc(jd, bd).start(add=True)

    for b in range(NBUF):
        s_desc(nj - NBUF + b, b).wait()
    plsc.subcore_barrier()
    pltpu.sync_copy(acc.at[pl.ds(sid * RPS, RPS)],
                    out_hbm.at[cid, pl.ds(sid * RPS, RPS)])


def _agg_call(hp, sd, zeros_blk):
    return pl.kernel(
        _agg_kernel,
        out_type=jax.ShapeDtypeStruct((NC, NPAD, HID), jnp.float32),
        mesh=_mesh,
        compiler_params=_SC_PARAMS,
        scratch_types=[
            pltpu.VMEM((CPT, 2, CHUNK), jnp.int32),
            pltpu.VMEM((NBUF, CHUNK, HID), jnp.float32),
            pltpu.VMEM_SHARED((NPAD, HID), jnp.float32),
            pltpu.SemaphoreType.DMA((NBUF,)),
            pltpu.SemaphoreType.DMA((NBUF,)),
        ],
    )(hp, sd, zeros_blk)


def _dinv(dp_ref):
    deg = dp_ref[0] + dp_ref[1] + 1.0
    return 1.0 / jnp.sqrt(deg[:, 0:1])


def _k1_body(x_ref, w_ref, dp_ref, hp_ref):
    hp_ref[...] = lax.dot_general(
        x_ref[...], w_ref[...], (((1,), (0,)), ((), ())), precision=_P
    ) * _dinv(dp_ref)


def _k2_body(p_ref, hp_ref, b_ref, dp_ref, w_ref, o_ref):
    dinv = _dinv(dp_ref)
    a = jnp.maximum(dinv * (p_ref[0] + p_ref[1] + hp_ref[...]) + b_ref[...], 0.0)
    o_ref[...] = lax.dot_general(
        a, w_ref[...], (((1,), (0,)), ((), ())), precision=_P
    ) * dinv


def _k4_body(p_ref, hp_ref, b_ref, dp_ref, batch_ref, wc_ref, bc_ref, o_ref):
    dinv = _dinv(dp_ref)
    a = jnp.maximum(dinv * (p_ref[0] + p_ref[1] + hp_ref[...]) + b_ref[...], 0.0)
    gid = lax.broadcasted_iota(jnp.int32, (NPAD, G), 1)
    oh = (batch_ref[...] == gid).astype(jnp.float32)
    sums = lax.dot_general(oh, a, (((0,), (0,)), ((), ())), precision=_P)
    cnts = lax.dot_general(oh, jnp.ones((NPAD, 1), jnp.float32),
                           (((0,), (0,)), ((), ())), precision=_P)
    pooled = sums / jnp.maximum(cnts, 1.0)
    o_ref[...] = lax.dot_general(
        pooled, wc_ref[...], (((1,), (0,)), ((), ())), precision=_P
    ) + bc_ref[...]


def kernel(x, edge_index, batch, W1, b1, W2, b2, W3, b3, Wc, bc):
    src = edge_index[0].astype(jnp.int32)
    dst = edge_index[1].astype(jnp.int32)
    srcf = jnp.concatenate([src, jnp.zeros((E_PAD - E,), jnp.int32)])
    dstf = jnp.concatenate([dst, jnp.full((E_PAD - E,), TRASH, jnp.int32)])
    e0 = NS * CPT0 * CHUNK

    def _group(flat):
        g0 = flat[:e0].reshape(NS, CPT0, CHUNK)
        g1 = flat[e0:].reshape(NS, CPT1, CHUNK)
        g0 = jnp.pad(g0, ((0, 0), (0, CPT - CPT0), (0, 0)))
        g1 = jnp.pad(g1, ((0, 0), (0, CPT - CPT1), (0, 0)))
        return jnp.concatenate([g0, g1], axis=0)

    sd = jnp.stack([_group(srcf), _group(dstf)], axis=2)
    xp = jnp.pad(x, ((0, NPAD - N), (0, 0)))
    batchp = jnp.pad(batch.astype(jnp.int32), (0, NPAD - N),
                     constant_values=G).reshape(NPAD, 1)
    z64 = jnp.zeros((RPS, HID), jnp.float32)
    zd = jnp.zeros((RPS, DW), jnp.float32)
    onesd = jnp.ones((CHUNK, DW), jnp.float32)
    b1r, b2r, b3r = b1.reshape(1, HID), b2.reshape(1, HID), b3.reshape(1, HID)
    bcr = bc.reshape(1, TASKS)

    dp = _deg_call(sd, onesd, zd)

    hp1 = pl.pallas_call(
        _k1_body, out_shape=jax.ShapeDtypeStruct((NPAD, HID), jnp.float32),
    )(xp, W1, dp)

    p1 = _agg_call(hp1, sd, z64)
    hp2 = pl.pallas_call(
        _k2_body, out_shape=jax.ShapeDtypeStruct((NPAD, HID), jnp.float32),
    )(p1, hp1, b1r, dp, W2)

    p2 = _agg_call(hp2, sd, z64)
    hp3 = pl.pallas_call(
        _k2_body, out_shape=jax.ShapeDtypeStruct((NPAD, HID), jnp.float32),
    )(p2, hp2, b2r, dp, W3)

    p3 = _agg_call(hp3, sd, z64)
    out = pl.pallas_call(
        _k4_body, out_shape=jax.ShapeDtypeStruct((G, TASKS), jnp.float32),
    )(p3, hp3, b3r, dp, batchp, Wc, bcr)
    return out


# speedup vs baseline: 1.1182x; 1.1182x over previous
import jax
import jax.numpy as jnp
from jax import lax
from jax.experimental import pallas as pl
from jax.experimental.pallas import tpu as pltpu
from jax.experimental.pallas import tpu_sc as plsc

N = 10000
E = 320000
IN_CH = 128
HID = 64
G = 64
TASKS = 2

NC, NS = 2, 16
NW = NC * NS
CHUNK = 128
NBUF = 6
DEPTH = 3
CPT0 = 126
CPT1 = 36
CPT = max(CPT0, CPT1)
E_PAD = NS * CHUNK * (CPT0 + CPT1)
NPAD = 10240
TRASH = N
RPS = NPAD // NS
DW = 16

_P = jax.lax.Precision.HIGHEST
_mesh = plsc.VectorSubcoreMesh(core_axis_name="c", subcore_axis_name="s")
_SC_PARAMS = pltpu.CompilerParams(use_tc_tiling_on_sc=False)


def _deg_kernel(sd_hbm, ones_hbm, z_hbm, out_hbm, idx_all, ones_v, dacc, sem):
    cid = lax.axis_index("c")
    sid = lax.axis_index("s")
    wid = cid * NS + sid
    nj = jnp.where(cid == 0, CPT0, CPT1)
    pltpu.sync_copy(z_hbm, dacc.at[pl.ds(sid * RPS, RPS)])
    pltpu.sync_copy(ones_hbm, ones_v)
    pltpu.sync_copy(sd_hbm.at[wid], idx_all)
    plsc.subcore_barrier()

    def s_desc(j, b):
        return pltpu.make_async_copy(
            ones_v, dacc.at[idx_all.at[j, 1]], sem.at[b])

    @pl.loop(0, CPT)
    def _(j):
        @pl.when(j < nj)
        def _():
            b = lax.rem(j, NBUF)

            @pl.when(j >= NBUF)
            def _():
                s_desc(j - NBUF, b).wait()

            s_desc(j, b).start(add=True)

    for b in range(NBUF):
        s_desc(nj - NBUF + b, b).wait()
    plsc.subcore_barrier()
    pltpu.sync_copy(dacc.at[pl.ds(sid * RPS, RPS)],
                    out_hbm.at[cid, pl.ds(sid * RPS, RPS)])


def _deg_call(sd, ones_blk, zeros_blk):
    return pl.kernel(
        _deg_kernel,
        out_type=jax.ShapeDtypeStruct((NC, NPAD, DW), jnp.float32),
        mesh=_mesh,
        compiler_params=_SC_PARAMS,
        scratch_types=[
            pltpu.VMEM((CPT, 2, CHUNK), jnp.int32),
            pltpu.VMEM((CHUNK, DW), jnp.float32),
            pltpu.VMEM_SHARED((NPAD, DW), jnp.float32),
            pltpu.SemaphoreType.DMA((NBUF,)),
        ],
    )(sd, ones_blk, zeros_blk)


def _agg_kernel(hp_hbm, sd_hbm, z_hbm, out_hbm, idx_all, rows, acc,
                semg, sems):
    cid = lax.axis_index("c")
    sid = lax.axis_index("s")
    wid = cid * NS + sid
    nj = jnp.where(cid == 0, CPT0, CPT1)
    sl = pl.ds(sid * RPS, RPS)
    pltpu.sync_copy(z_hbm, acc.at[sl])
    pltpu.sync_copy(sd_hbm.at[wid], idx_all)
    plsc.subcore_barrier()

    def g_desc(j, b):
        return pltpu.make_async_copy(
            hp_hbm.at[idx_all.at[j, 0]], rows.at[b], semg.at[b])

    def s_desc(j, b):
        return pltpu.make_async_copy(
            rows.at[b], acc.at[idx_all.at[j, 1]], sems.at[b])

    @pl.loop(0, CPT + DEPTH)
    def _(j):
        @pl.when(j < nj)
        def _():
            b = lax.rem(j, NBUF)

            @pl.when(j >= NBUF)
            def _():
                s_desc(j - NBUF, b).wait()

            g_desc(j, b).start()

        @pl.when(jnp.logical_and(j >= DEPTH, j < nj + DEPTH))
        def _():
            jd = j - DEPTH
            bd = lax.rem(jd, NBUF)
            g_desc(jd, bd).wait()
            s_desc(jd, bd).start(add=True)

    for b in range(NBUF):
        s_desc(nj - NBUF + b, b).wait()
    plsc.subcore_barrier()
    pltpu.sync_copy(acc.at[pl.ds(sid * RPS, RPS)],
                    out_hbm.at[cid, pl.ds(sid * RPS, RPS)])


def _agg_call(hp, sd, zeros_blk):
    return pl.kernel(
        _agg_kernel,
        out_type=jax.ShapeDtypeStruct((NC, NPAD, HID), jnp.float32),
        mesh=_mesh,
        compiler_params=_SC_PARAMS,
        scratch_types=[
            pltpu.VMEM((CPT, 2, CHUNK), jnp.int32),
            pltpu.VMEM((NBUF, CHUNK, HID), jnp.float32),
            pltpu.VMEM_SHARED((NPAD, HID), jnp.float32),
            pltpu.SemaphoreType.DMA((NBUF,)),
            pltpu.SemaphoreType.DMA((NBUF,)),
        ],
    )(hp, sd, zeros_blk)


def _dinv(dp_ref):
    deg = dp_ref[0] + dp_ref[1] + 1.0
    return 1.0 / jnp.sqrt(deg[:, 0:1])


def _k1_body(x_ref, w_ref, dp_ref, hp_ref):
    hp_ref[...] = lax.dot_general(
        x_ref[...], w_ref[...], (((1,), (0,)), ((), ())), precision=_P
    ) * _dinv(dp_ref)


def _k2_body(p_ref, hp_ref, b_ref, dp_ref, w_ref, o_ref):
    dinv = _dinv(dp_ref)
    a = jnp.maximum(dinv * (p_ref[0] + p_ref[1] + hp_ref[...]) + b_ref[...], 0.0)
    o_ref[...] = lax.dot_general(
        a, w_ref[...], (((1,), (0,)), ((), ())), precision=_P
    ) * dinv


def _k4_body(p_ref, hp_ref, b_ref, dp_ref, batch_ref, wc_ref, bc_ref, o_ref):
    dinv = _dinv(dp_ref)
    a = jnp.maximum(dinv * (p_ref[0] + p_ref[1] + hp_ref[...]) + b_ref[...], 0.0)
    gid = lax.broadcasted_iota(jnp.int32, (NPAD, G), 1)
    oh = (batch_ref[...] == gid).astype(jnp.float32)
    sums = lax.dot_general(oh, a, (((0,), (0,)), ((), ())), precision=_P)
    cnts = lax.dot_general(oh, jnp.ones((NPAD, 1), jnp.float32),
                           (((0,), (0,)), ((), ())), precision=_P)
    pooled = sums / jnp.maximum(cnts, 1.0)
    o_ref[...] = lax.dot_general(
        pooled, wc_ref[...], (((1,), (0,)), ((), ())), precision=_P
    ) + bc_ref[...]


def kernel(x, edge_index, batch, W1, b1, W2, b2, W3, b3, Wc, bc):
    src = edge_index[0].astype(jnp.int32)
    dst = edge_index[1].astype(jnp.int32)
    srcf = jnp.concatenate([src, jnp.zeros((E_PAD - E,), jnp.int32)])
    dstf = jnp.concatenate([dst, jnp.full((E_PAD - E,), TRASH, jnp.int32)])
    e0 = NS * CPT0 * CHUNK

    def _group(flat):
        g0 = flat[:e0].reshape(NS, CPT0, CHUNK)
        g1 = flat[e0:].reshape(NS, CPT1, CHUNK)
        g0 = jnp.pad(g0, ((0, 0), (0, CPT - CPT0), (0, 0)))
        g1 = jnp.pad(g1, ((0, 0), (0, CPT - CPT1), (0, 0)))
        return jnp.concatenate([g0, g1], axis=0)

    sd = jnp.stack([_group(srcf), _group(dstf)], axis=2)
    xp = jnp.pad(x, ((0, NPAD - N), (0, 0)))
    batchp = jnp.pad(batch.astype(jnp.int32), (0, NPAD - N),
                     constant_values=G).reshape(NPAD, 1)
    z64 = jnp.zeros((RPS, HID), jnp.float32)
    zd = jnp.zeros((RPS, DW), jnp.float32)
    onesd = jnp.ones((CHUNK, DW), jnp.float32)
    b1r, b2r, b3r = b1.reshape(1, HID), b2.reshape(1, HID), b3.reshape(1, HID)
    bcr = bc.reshape(1, TASKS)

    dp = _deg_call(sd, onesd, zd)

    hp1 = pl.pallas_call(
        _k1_body, out_shape=jax.ShapeDtypeStruct((NPAD, HID), jnp.float32),
    )(xp, W1, dp)

    p1 = _agg_call(hp1, sd, z64)
    hp2 = pl.pallas_call(
        _k2_body, out_shape=jax.ShapeDtypeStruct((NPAD, HID), jnp.float32),
    )(p1, hp1, b1r, dp, W2)

    p2 = _agg_call(hp2, sd, z64)
    hp3 = pl.pallas_call(
        _k2_body, out_shape=jax.ShapeDtypeStruct((NPAD, HID), jnp.float32),
    )(p2, hp2, b2r, dp, W3)

    p3 = _agg_call(hp3, sd, z64)
    out = pl.pallas_call(
        _k4_body, out_shape=jax.ShapeDtypeStruct((G, TASKS), jnp.float32),
    )(p3, hp3, b3r, dp, batchp, Wc, bcr)
    return out


# speedup vs baseline: 3.4797x; 3.1119x over previous
import jax
import jax.numpy as jnp
from jax import lax
from jax.experimental import pallas as pl
from jax.experimental.pallas import tpu as pltpu
from jax.experimental.pallas import tpu_sc as plsc

N = 10000
E = 320000
IN_CH = 128
HID = 64
G = 64
TASKS = 2

NC, NS = 2, 16
NW = NC * NS
CHUNK = 128
NBUF = 8
DEPTH = 4
CPT0 = 80
CPT1 = 80
CPT = max(CPT0, CPT1)
E_PAD = NS * CHUNK * (CPT0 + CPT1)
NPAD = 10240
TRASH = N
RPS = NPAD // NS
DW = 16

_P = jax.lax.Precision.HIGHEST
_mesh = plsc.VectorSubcoreMesh(core_axis_name="c", subcore_axis_name="s")
_SC_PARAMS = pltpu.CompilerParams(use_tc_tiling_on_sc=False)


def _deg_kernel(sd_hbm, ones_hbm, z_hbm, out_hbm, idx_all, ones_v, dacc, sem):
    cid = lax.axis_index("c")
    sid = lax.axis_index("s")
    wid = cid * NS + sid
    nj = jnp.where(cid == 0, CPT0, CPT1)
    pltpu.sync_copy(z_hbm, dacc.at[pl.ds(sid * RPS, RPS)])
    pltpu.sync_copy(ones_hbm, ones_v)
    pltpu.sync_copy(sd_hbm.at[wid], idx_all)
    plsc.subcore_barrier()

    def s_desc(j, b):
        return pltpu.make_async_copy(
            ones_v, dacc.at[idx_all.at[j, 1]], sem.at[b])

    @pl.loop(0, CPT)
    def _(j):
        @pl.when(j < nj)
        def _():
            b = lax.rem(j, NBUF)

            @pl.when(j >= NBUF)
            def _():
                s_desc(j - NBUF, b).wait()

            s_desc(j, b).start(add=True)

    for b in range(NBUF):
        s_desc(nj - NBUF + b, b).wait()
    plsc.subcore_barrier()
    pltpu.sync_copy(dacc.at[pl.ds(sid * RPS, RPS)],
                    out_hbm.at[cid, pl.ds(sid * RPS, RPS)])


def _deg_call(sd, ones_blk, zeros_blk):
    return pl.kernel(
        _deg_kernel,
        out_type=jax.ShapeDtypeStruct((NC, NPAD, DW), jnp.float32),
        mesh=_mesh,
        compiler_params=_SC_PARAMS,
        scratch_types=[
            pltpu.VMEM((CPT, 2, CHUNK), jnp.int32),
            pltpu.VMEM((CHUNK, DW), jnp.float32),
            pltpu.VMEM_SHARED((NPAD, DW), jnp.float32),
            pltpu.SemaphoreType.DMA((NBUF,)),
        ],
    )(sd, ones_blk, zeros_blk)


def _agg_kernel(hp_hbm, sd_hbm, z_hbm, out_hbm, idx_all, rows, acc,
                semg, sems):
    cid = lax.axis_index("c")
    sid = lax.axis_index("s")
    wid = cid * NS + sid
    nj = jnp.where(cid == 0, CPT0, CPT1)
    sl = pl.ds(sid * RPS, RPS)
    pltpu.sync_copy(z_hbm, acc.at[sl])
    pltpu.sync_copy(sd_hbm.at[wid], idx_all)
    plsc.subcore_barrier()

    def g_desc(j, b):
        return pltpu.make_async_copy(
            hp_hbm.at[idx_all.at[j, 0]], rows.at[b], semg.at[b])

    def s_desc(j, b):
        return pltpu.make_async_copy(
            rows.at[b], acc.at[idx_all.at[j, 1]], sems.at[b])

    @pl.loop(0, CPT + DEPTH)
    def _(j):
        @pl.when(j < nj)
        def _():
            b = lax.rem(j, NBUF)

            @pl.when(j >= NBUF)
            def _():
                s_desc(j - NBUF, b).wait()

            g_desc(j, b).start()

        @pl.when(jnp.logical_and(j >= DEPTH, j < nj + DEPTH))
        def _():
            jd = j - DEPTH
            bd = lax.rem(jd, NBUF)
            g_desc(jd, bd).wait()
            s_desc(jd, bd).start(add=True)

    for b in range(NBUF):
        s_desc(nj - NBUF + b, b).wait()
    plsc.subcore_barrier()
    pltpu.sync_copy(acc.at[pl.ds(sid * RPS, RPS)],
                    out_hbm.at[cid, pl.ds(sid * RPS, RPS)])


def _agg_call(hp, sd, zeros_blk):
    return pl.kernel(
        _agg_kernel,
        out_type=jax.ShapeDtypeStruct((NC, NPAD, HID), jnp.float32),
        mesh=_mesh,
        compiler_params=_SC_PARAMS,
        scratch_types=[
            pltpu.VMEM((CPT, 2, CHUNK), jnp.int32),
            pltpu.VMEM((NBUF, CHUNK, HID), jnp.float32),
            pltpu.VMEM_SHARED((NPAD, HID), jnp.float32),
            pltpu.SemaphoreType.DMA((NBUF,)),
            pltpu.SemaphoreType.DMA((NBUF,)),
        ],
    )(hp, sd, zeros_blk)


def _dinv(dp_ref):
    deg = dp_ref[0] + dp_ref[1] + 1.0
    return 1.0 / jnp.sqrt(deg[:, 0:1])


def _k1_body(x_ref, w_ref, dp_ref, hp_ref):
    hp_ref[...] = lax.dot_general(
        x_ref[...], w_ref[...], (((1,), (0,)), ((), ())), precision=_P
    ) * _dinv(dp_ref)


def _k2_body(p_ref, hp_ref, b_ref, dp_ref, w_ref, o_ref):
    dinv = _dinv(dp_ref)
    a = jnp.maximum(dinv * (p_ref[0] + p_ref[1] + hp_ref[...]) + b_ref[...], 0.0)
    o_ref[...] = lax.dot_general(
        a, w_ref[...], (((1,), (0,)), ((), ())), precision=_P
    ) * dinv


def _k4_body(p_ref, hp_ref, b_ref, dp_ref, batch_ref, wc_ref, bc_ref, o_ref):
    dinv = _dinv(dp_ref)
    a = jnp.maximum(dinv * (p_ref[0] + p_ref[1] + hp_ref[...]) + b_ref[...], 0.0)
    gid = lax.broadcasted_iota(jnp.int32, (NPAD, G), 1)
    oh = (batch_ref[...] == gid).astype(jnp.float32)
    sums = lax.dot_general(oh, a, (((0,), (0,)), ((), ())), precision=_P)
    cnts = lax.dot_general(oh, jnp.ones((NPAD, 1), jnp.float32),
                           (((0,), (0,)), ((), ())), precision=_P)
    pooled = sums / jnp.maximum(cnts, 1.0)
    o_ref[...] = lax.dot_general(
        pooled, wc_ref[...], (((1,), (0,)), ((), ())), precision=_P
    ) + bc_ref[...]


def kernel(x, edge_index, batch, W1, b1, W2, b2, W3, b3, Wc, bc):
    src = edge_index[0].astype(jnp.int32)
    dst = edge_index[1].astype(jnp.int32)
    padi = jnp.arange(E_PAD - E, dtype=jnp.int32)
    srcf = jnp.concatenate([src, padi % N])
    dstf = jnp.concatenate([dst, TRASH + padi % (NPAD - N)])
    e0 = NS * CPT0 * CHUNK

    def _group(flat):
        g0 = flat[:e0].reshape(NS, CPT0, CHUNK)
        g1 = flat[e0:].reshape(NS, CPT1, CHUNK)
        g0 = jnp.pad(g0, ((0, 0), (0, CPT - CPT0), (0, 0)))
        g1 = jnp.pad(g1, ((0, 0), (0, CPT - CPT1), (0, 0)))
        return jnp.concatenate([g0, g1], axis=0)

    sd = jnp.stack([_group(srcf), _group(dstf)], axis=2)
    xp = jnp.pad(x, ((0, NPAD - N), (0, 0)))
    batchp = jnp.pad(batch.astype(jnp.int32), (0, NPAD - N),
                     constant_values=G).reshape(NPAD, 1)
    z64 = jnp.zeros((RPS, HID), jnp.float32)
    zd = jnp.zeros((RPS, DW), jnp.float32)
    onesd = jnp.ones((CHUNK, DW), jnp.float32)
    b1r, b2r, b3r = b1.reshape(1, HID), b2.reshape(1, HID), b3.reshape(1, HID)
    bcr = bc.reshape(1, TASKS)

    dp = _deg_call(sd, onesd, zd)

    hp1 = pl.pallas_call(
        _k1_body, out_shape=jax.ShapeDtypeStruct((NPAD, HID), jnp.float32),
    )(xp, W1, dp)

    p1 = _agg_call(hp1, sd, z64)
    hp2 = pl.pallas_call(
        _k2_body, out_shape=jax.ShapeDtypeStruct((NPAD, HID), jnp.float32),
    )(p1, hp1, b1r, dp, W2)

    p2 = _agg_call(hp2, sd, z64)
    hp3 = pl.pallas_call(
        _k2_body, out_shape=jax.ShapeDtypeStruct((NPAD, HID), jnp.float32),
    )(p2, hp2, b2r, dp, W3)

    p3 = _agg_call(hp3, sd, z64)
    out = pl.pallas_call(
        _k4_body, out_shape=jax.ShapeDtypeStruct((G, TASKS), jnp.float32),
    )(p3, hp3, b3r, dp, batchp, Wc, bcr)
    return out


# speedup vs baseline: 3.5372x; 1.0165x over previous
import jax
import jax.numpy as jnp
from jax import lax
from jax.experimental import pallas as pl
from jax.experimental.pallas import tpu as pltpu
from jax.experimental.pallas import tpu_sc as plsc

N = 10000
E = 320000
IN_CH = 128
HID = 64
G = 64
TASKS = 2

NC, NS = 2, 16
NW = NC * NS
CHUNK = 128
NBUF = 8
DEPTH = 4
NCHUNK = E // CHUNK
CPQ = NCHUNK // NW
CPR = NCHUNK % NW
CPT = CPQ + 1
E2 = (NCHUNK + 1) * CHUNK
NPAD = 10240
RPS = NPAD // NS
DW = 16

_P = jax.lax.Precision.HIGHEST
_mesh = plsc.VectorSubcoreMesh(core_axis_name="c", subcore_axis_name="s")
_SC_PARAMS = pltpu.CompilerParams(use_tc_tiling_on_sc=False)


def _tile_work(cid, sid):
    wid = cid * NS + sid
    nj = jnp.where(wid < CPR, CPQ + 1, CPQ)
    cbase = CPQ * wid + jnp.minimum(wid, CPR)
    return wid, nj, cbase


def _deg_kernel(dstc_hbm, ones_hbm, z_hbm, out_hbm, dst_all, ones_v, dacc,
                sem):
    cid = lax.axis_index("c")
    sid = lax.axis_index("s")
    wid, nj, cbase = _tile_work(cid, sid)
    pltpu.sync_copy(z_hbm, dacc.at[pl.ds(sid * RPS, RPS)])
    pltpu.sync_copy(ones_hbm, ones_v)
    pltpu.sync_copy(dstc_hbm.at[pl.ds(cbase, CPT)], dst_all)
    plsc.subcore_barrier()

    def s_desc(j, b):
        return pltpu.make_async_copy(
            ones_v, dacc.at[dst_all.at[j]], sem.at[b])

    @pl.loop(0, CPT)
    def _(j):
        @pl.when(j < nj)
        def _():
            b = lax.rem(j, NBUF)

            @pl.when(j >= NBUF)
            def _():
                s_desc(j - NBUF, b).wait()

            s_desc(j, b).start(add=True)

    for b in range(NBUF):
        jl = nj - NBUF + b
        s_desc(jl, lax.rem(jl, NBUF)).wait()
    plsc.subcore_barrier()
    pltpu.sync_copy(dacc.at[pl.ds(sid * RPS, RPS)],
                    out_hbm.at[cid, pl.ds(sid * RPS, RPS)])


def _deg_call(dstc, ones_blk, zeros_blk):
    return pl.kernel(
        _deg_kernel,
        out_type=jax.ShapeDtypeStruct((NC, NPAD, DW), jnp.float32),
        mesh=_mesh,
        compiler_params=_SC_PARAMS,
        scratch_types=[
            pltpu.VMEM((CPT, CHUNK), jnp.int32),
            pltpu.VMEM((CHUNK, DW), jnp.float32),
            pltpu.VMEM_SHARED((NPAD, DW), jnp.float32),
            pltpu.SemaphoreType.DMA((NBUF,)),
        ],
    )(dstc, ones_blk, zeros_blk)


def _agg_kernel(hp_hbm, srcc_hbm, dstc_hbm, z_hbm, out_hbm, src_all, dst_all,
                rows, acc, semg, sems):
    cid = lax.axis_index("c")
    sid = lax.axis_index("s")
    wid, nj, cbase = _tile_work(cid, sid)
    sl = pl.ds(sid * RPS, RPS)
    pltpu.sync_copy(z_hbm, acc.at[sl])
    pltpu.sync_copy(srcc_hbm.at[pl.ds(cbase, CPT)], src_all)
    pltpu.sync_copy(dstc_hbm.at[pl.ds(cbase, CPT)], dst_all)
    plsc.subcore_barrier()

    def g_desc(j, b):
        return pltpu.make_async_copy(
            hp_hbm.at[src_all.at[j]], rows.at[b], semg.at[b])

    def s_desc(j, b):
        return pltpu.make_async_copy(
            rows.at[b], acc.at[dst_all.at[j]], sems.at[b])

    @pl.loop(0, CPT + DEPTH)
    def _(j):
        @pl.when(j < nj)
        def _():
            b = lax.rem(j, NBUF)

            @pl.when(j >= NBUF)
            def _():
                s_desc(j - NBUF, b).wait()

            g_desc(j, b).start()

        @pl.when(jnp.logical_and(j >= DEPTH, j < nj + DEPTH))
        def _():
            jd = j - DEPTH
            bd = lax.rem(jd, NBUF)
            g_desc(jd, bd).wait()
            s_desc(jd, bd).start(add=True)

    for b in range(NBUF):
        jl = nj - NBUF + b
        s_desc(jl, lax.rem(jl, NBUF)).wait()
    plsc.subcore_barrier()
    pltpu.sync_copy(acc.at[pl.ds(sid * RPS, RPS)],
                    out_hbm.at[cid, pl.ds(sid * RPS, RPS)])


def _agg_call(hp, srcc, dstc, zeros_blk):
    return pl.kernel(
        _agg_kernel,
        out_type=jax.ShapeDtypeStruct((NC, NPAD, HID), jnp.float32),
        mesh=_mesh,
        compiler_params=_SC_PARAMS,
        scratch_types=[
            pltpu.VMEM((CPT, CHUNK), jnp.int32),
            pltpu.VMEM((CPT, CHUNK), jnp.int32),
            pltpu.VMEM((NBUF, CHUNK, HID), jnp.float32),
            pltpu.VMEM_SHARED((NPAD, HID), jnp.float32),
            pltpu.SemaphoreType.DMA((NBUF,)),
            pltpu.SemaphoreType.DMA((NBUF,)),
        ],
    )(hp, srcc, dstc, zeros_blk)


def _dinv(dp_ref):
    deg = dp_ref[0] + dp_ref[1] + 1.0
    return 1.0 / jnp.sqrt(deg[:, 0:1])


def _k1_body(x_ref, w_ref, dp_ref, hp_ref):
    hp_ref[...] = lax.dot_general(
        x_ref[...], w_ref[...], (((1,), (0,)), ((), ())), precision=_P
    ) * _dinv(dp_ref)


def _k2_body(p_ref, hp_ref, b_ref, dp_ref, w_ref, o_ref):
    dinv = _dinv(dp_ref)
    a = jnp.maximum(dinv * (p_ref[0] + p_ref[1] + hp_ref[...]) + b_ref[...], 0.0)
    o_ref[...] = lax.dot_general(
        a, w_ref[...], (((1,), (0,)), ((), ())), precision=_P
    ) * dinv


def _k4_body(p_ref, hp_ref, b_ref, dp_ref, batch_ref, wc_ref, bc_ref, o_ref):
    dinv = _dinv(dp_ref)
    a = jnp.maximum(dinv * (p_ref[0] + p_ref[1] + hp_ref[...]) + b_ref[...], 0.0)
    gid = lax.broadcasted_iota(jnp.int32, (NPAD, G), 1)
    oh = (batch_ref[...] == gid).astype(jnp.float32)
    sums = lax.dot_general(oh, a, (((0,), (0,)), ((), ())), precision=_P)
    cnts = lax.dot_general(oh, jnp.ones((NPAD, 1), jnp.float32),
                           (((0,), (0,)), ((), ())), precision=_P)
    pooled = sums / jnp.maximum(cnts, 1.0)
    o_ref[...] = lax.dot_general(
        pooled, wc_ref[...], (((1,), (0,)), ((), ())), precision=_P
    ) + bc_ref[...]


def kernel(x, edge_index, batch, W1, b1, W2, b2, W3, b3, Wc, bc):
    src = edge_index[0].astype(jnp.int32)
    dst = edge_index[1].astype(jnp.int32)
    srcc = jnp.pad(src, (0, E2 - E)).reshape(E2 // CHUNK, CHUNK)
    dstc = jnp.pad(dst, (0, E2 - E)).reshape(E2 // CHUNK, CHUNK)
    xp = jnp.pad(x, ((0, NPAD - N), (0, 0)))
    batchp = jnp.pad(batch.astype(jnp.int32), (0, NPAD - N),
                     constant_values=G).reshape(NPAD, 1)
    z64 = jnp.zeros((RPS, HID), jnp.float32)
    zd = jnp.zeros((RPS, DW), jnp.float32)
    onesd = jnp.ones((CHUNK, DW), jnp.float32)
    b1r, b2r, b3r = b1.reshape(1, HID), b2.reshape(1, HID), b3.reshape(1, HID)
    bcr = bc.reshape(1, TASKS)

    dp = _deg_call(dstc, onesd, zd)

    hp1 = pl.pallas_call(
        _k1_body, out_shape=jax.ShapeDtypeStruct((NPAD, HID), jnp.float32),
    )(xp, W1, dp)

    p1 = _agg_call(hp1, srcc, dstc, z64)
    hp2 = pl.pallas_call(
        _k2_body, out_shape=jax.ShapeDtypeStruct((NPAD, HID), jnp.float32),
    )(p1, hp1, b1r, dp, W2)

    p2 = _agg_call(hp2, srcc, dstc, z64)
    hp3 = pl.pallas_call(
        _k2_body, out_shape=jax.ShapeDtypeStruct((NPAD, HID), jnp.float32),
    )(p2, hp2, b2r, dp, W3)

    p3 = _agg_call(hp3, srcc, dstc, z64)
    out = pl.pallas_call(
        _k4_body, out_shape=jax.ShapeDtypeStruct((G, TASKS), jnp.float32),
    )(p3, hp3, b3r, dp, batchp, Wc, bcr)
    return out


# speedup vs baseline: 3.6421x; 1.0297x over previous
import jax
import jax.numpy as jnp
from jax import lax
from jax.experimental import pallas as pl
from jax.experimental.pallas import tpu as pltpu
from jax.experimental.pallas import tpu_sc as plsc

N = 10000
E = 320000
IN_CH = 128
HID = 64
G = 64
TASKS = 2

NC, NS = 2, 16
NW = NC * NS
CHUNK = 128
NBUF = 8
DEPTH = 4
NCHUNK = E // CHUNK
CPQ = NCHUNK // NW
CPR = NCHUNK % NW
CPT = CPQ + 1
NPAD = 10240
RPS = NPAD // NS
DW = 16

_P = jax.lax.Precision.HIGHEST
_mesh = plsc.VectorSubcoreMesh(core_axis_name="c", subcore_axis_name="s")
_SC_PARAMS = pltpu.CompilerParams(use_tc_tiling_on_sc=False)


def _tile_work(cid, sid):
    wid = cid * NS + sid
    nj = jnp.where(wid < CPR, CPQ + 1, CPQ)
    cbase = CPQ * wid + jnp.minimum(wid, CPR)
    off = jnp.where(cbase + CPT > NCHUNK, 1, 0)
    return wid, nj, cbase - off, off


def _deg_kernel(ei_hbm, ones_hbm, z_hbm, out_hbm, dst_all, ones_v, dacc,
                sem):
    cid = lax.axis_index("c")
    sid = lax.axis_index("s")
    wid, nj, cbase, off = _tile_work(cid, sid)
    pltpu.sync_copy(z_hbm, dacc.at[pl.ds(sid * RPS, RPS)])
    pltpu.sync_copy(ones_hbm, ones_v)
    pltpu.sync_copy(ei_hbm.at[1, pl.ds(cbase, CPT)], dst_all)
    plsc.subcore_barrier()

    def s_desc(j, b):
        return pltpu.make_async_copy(
            ones_v, dacc.at[dst_all.at[j + off]], sem.at[b])

    @pl.loop(0, CPT)
    def _(j):
        @pl.when(j < nj)
        def _():
            b = lax.rem(j, NBUF)

            @pl.when(j >= NBUF)
            def _():
                s_desc(j - NBUF, b).wait()

            s_desc(j, b).start(add=True)

    for b in range(NBUF):
        jl = nj - NBUF + b
        s_desc(jl, lax.rem(jl, NBUF)).wait()
    plsc.subcore_barrier()
    pltpu.sync_copy(dacc.at[pl.ds(sid * RPS, RPS)],
                    out_hbm.at[cid, pl.ds(sid * RPS, RPS)])


def _deg_call(ei3, ones_blk, zeros_blk):
    return pl.kernel(
        _deg_kernel,
        out_type=jax.ShapeDtypeStruct((NC, NPAD, DW), jnp.float32),
        mesh=_mesh,
        compiler_params=_SC_PARAMS,
        scratch_types=[
            pltpu.VMEM((CPT, CHUNK), jnp.int32),
            pltpu.VMEM((CHUNK, DW), jnp.float32),
            pltpu.VMEM_SHARED((NPAD, DW), jnp.float32),
            pltpu.SemaphoreType.DMA((NBUF,)),
        ],
    )(ei3, ones_blk, zeros_blk)


def _agg_kernel(hp_hbm, ei_hbm, z_hbm, out_hbm, src_all, dst_all,
                rows, acc, semg, sems):
    cid = lax.axis_index("c")
    sid = lax.axis_index("s")
    wid, nj, cbase, off = _tile_work(cid, sid)
    sl = pl.ds(sid * RPS, RPS)
    pltpu.sync_copy(z_hbm, acc.at[sl])
    pltpu.sync_copy(ei_hbm.at[0, pl.ds(cbase, CPT)], src_all)
    pltpu.sync_copy(ei_hbm.at[1, pl.ds(cbase, CPT)], dst_all)
    plsc.subcore_barrier()

    def g_desc(j, b):
        return pltpu.make_async_copy(
            hp_hbm.at[src_all.at[j + off]], rows.at[b], semg.at[b])

    def s_desc(j, b):
        return pltpu.make_async_copy(
            rows.at[b], acc.at[dst_all.at[j + off]], sems.at[b])

    @pl.loop(0, CPT + DEPTH)
    def _(j):
        @pl.when(j < nj)
        def _():
            b = lax.rem(j, NBUF)

            @pl.when(j >= NBUF)
            def _():
                s_desc(j - NBUF, b).wait()

            g_desc(j, b).start()

        @pl.when(jnp.logical_and(j >= DEPTH, j < nj + DEPTH))
        def _():
            jd = j - DEPTH
            bd = lax.rem(jd, NBUF)
            g_desc(jd, bd).wait()
            s_desc(jd, bd).start(add=True)

    for b in range(NBUF):
        jl = nj - NBUF + b
        s_desc(jl, lax.rem(jl, NBUF)).wait()
    plsc.subcore_barrier()
    pltpu.sync_copy(acc.at[pl.ds(sid * RPS, RPS)],
                    out_hbm.at[cid, pl.ds(sid * RPS, RPS)])


def _agg_call(hp, ei3, zeros_blk):
    return pl.kernel(
        _agg_kernel,
        out_type=jax.ShapeDtypeStruct((NC, NPAD, HID), jnp.float32),
        mesh=_mesh,
        compiler_params=_SC_PARAMS,
        scratch_types=[
            pltpu.VMEM((CPT, CHUNK), jnp.int32),
            pltpu.VMEM((CPT, CHUNK), jnp.int32),
            pltpu.VMEM((NBUF, CHUNK, HID), jnp.float32),
            pltpu.VMEM_SHARED((NPAD, HID), jnp.float32),
            pltpu.SemaphoreType.DMA((NBUF,)),
            pltpu.SemaphoreType.DMA((NBUF,)),
        ],
    )(hp, ei3, zeros_blk)


def _dinv(dp_ref):
    deg = dp_ref[0] + dp_ref[1] + 1.0
    return 1.0 / jnp.sqrt(deg[:, 0:1])


def _k1a_body(x_ref, w_ref, u_ref):
    u_ref[...] = lax.dot_general(
        x_ref[...], w_ref[...], (((1,), (0,)), ((), ())), precision=_P)


def _k1b_body(u_ref, dp_ref, hp_ref):
    hp_ref[...] = u_ref[...] * _dinv(dp_ref)


def _k2_body(p_ref, hp_ref, b_ref, dp_ref, w_ref, o_ref):
    dinv = _dinv(dp_ref)
    a = jnp.maximum(dinv * (p_ref[0] + p_ref[1] + hp_ref[...]) + b_ref[...], 0.0)
    o_ref[...] = lax.dot_general(
        a, w_ref[...], (((1,), (0,)), ((), ())), precision=_P
    ) * dinv


def _k4_body(p_ref, hp_ref, b_ref, dp_ref, batch_ref, wc_ref, bc_ref, o_ref):
    dinv = _dinv(dp_ref)
    a = jnp.maximum(dinv * (p_ref[0] + p_ref[1] + hp_ref[...]) + b_ref[...], 0.0)
    gid = lax.broadcasted_iota(jnp.int32, (NPAD, G), 1)
    oh = (batch_ref[...] == gid).astype(jnp.float32)
    sums = lax.dot_general(oh, a, (((0,), (0,)), ((), ())), precision=_P)
    cnts = lax.dot_general(oh, jnp.ones((NPAD, 1), jnp.float32),
                           (((0,), (0,)), ((), ())), precision=_P)
    pooled = sums / jnp.maximum(cnts, 1.0)
    o_ref[...] = lax.dot_general(
        pooled, wc_ref[...], (((1,), (0,)), ((), ())), precision=_P
    ) + bc_ref[...]


def kernel(x, edge_index, batch, W1, b1, W2, b2, W3, b3, Wc, bc):
    ei3 = edge_index.astype(jnp.int32).reshape(2, NCHUNK, CHUNK)
    xp = jnp.pad(x, ((0, NPAD - N), (0, 0)))
    batchp = jnp.pad(batch.astype(jnp.int32), (0, NPAD - N),
                     constant_values=G).reshape(NPAD, 1)
    z64 = jnp.zeros((RPS, HID), jnp.float32)
    zd = jnp.zeros((RPS, DW), jnp.float32)
    onesd = jnp.ones((CHUNK, DW), jnp.float32)
    b1r, b2r, b3r = b1.reshape(1, HID), b2.reshape(1, HID), b3.reshape(1, HID)
    bcr = bc.reshape(1, TASKS)

    dp = _deg_call(ei3, onesd, zd)

    u1 = pl.pallas_call(
        _k1a_body, out_shape=jax.ShapeDtypeStruct((NPAD, HID), jnp.float32),
    )(xp, W1)
    hp1 = pl.pallas_call(
        _k1b_body, out_shape=jax.ShapeDtypeStruct((NPAD, HID), jnp.float32),
    )(u1, dp)

    p1 = _agg_call(hp1, ei3, z64)
    hp2 = pl.pallas_call(
        _k2_body, out_shape=jax.ShapeDtypeStruct((NPAD, HID), jnp.float32),
    )(p1, hp1, b1r, dp, W2)

    p2 = _agg_call(hp2, ei3, z64)
    hp3 = pl.pallas_call(
        _k2_body, out_shape=jax.ShapeDtypeStruct((NPAD, HID), jnp.float32),
    )(p2, hp2, b2r, dp, W3)

    p3 = _agg_call(hp3, ei3, z64)
    out = pl.pallas_call(
        _k4_body, out_shape=jax.ShapeDtypeStruct((G, TASKS), jnp.float32),
    )(p3, hp3, b3r, dp, batchp, Wc, bcr)
    return out


# speedup vs baseline: 3.7191x; 1.0211x over previous
import jax
import jax.numpy as jnp
from jax import lax
from jax.experimental import pallas as pl
from jax.experimental.pallas import tpu as pltpu
from jax.experimental.pallas import tpu_sc as plsc

N = 10000
E = 320000
IN_CH = 128
HID = 64
G = 64
TASKS = 2

NC, NS = 2, 16
NW = NC * NS
CHUNK = 128
NBUF = 8
DEPTH = 6
NCHUNK = E // CHUNK
CPQ = NCHUNK // NW
CPR = NCHUNK % NW
CPT = CPQ + 1
NPAD = 10240
RPS = NPAD // NS
DW = 16

_P = jax.lax.Precision.HIGHEST
_mesh = plsc.VectorSubcoreMesh(core_axis_name="c", subcore_axis_name="s")
_SC_PARAMS = pltpu.CompilerParams(use_tc_tiling_on_sc=False)


def _tile_work(cid, sid):
    wid = cid * NS + sid
    nj = jnp.where(wid < CPR, CPQ + 1, CPQ)
    cbase = CPQ * wid + jnp.minimum(wid, CPR)
    off = jnp.where(cbase + CPT > NCHUNK, 1, 0)
    return wid, nj, cbase - off, off


def _deg_kernel(ei_hbm, ones_hbm, z_hbm, out_hbm, dst_all, ones_v, dacc,
                sem):
    cid = lax.axis_index("c")
    sid = lax.axis_index("s")
    wid, nj, cbase, off = _tile_work(cid, sid)
    pltpu.sync_copy(z_hbm, dacc.at[pl.ds(sid * RPS, RPS)])
    pltpu.sync_copy(ones_hbm, ones_v)
    pltpu.sync_copy(ei_hbm.at[1, pl.ds(cbase, CPT)], dst_all)
    plsc.subcore_barrier()

    def s_desc(j, b):
        return pltpu.make_async_copy(
            ones_v, dacc.at[dst_all.at[j + off]], sem.at[b])

    @pl.loop(0, CPT)
    def _(j):
        @pl.when(j < nj)
        def _():
            b = lax.rem(j, NBUF)

            @pl.when(j >= NBUF)
            def _():
                s_desc(j - NBUF, b).wait()

            s_desc(j, b).start(add=True)

    for b in range(NBUF):
        jl = nj - NBUF + b
        s_desc(jl, lax.rem(jl, NBUF)).wait()
    plsc.subcore_barrier()
    pltpu.sync_copy(dacc.at[pl.ds(sid * RPS, RPS)],
                    out_hbm.at[cid, pl.ds(sid * RPS, RPS)])


def _deg_call(ei3, ones_blk, zeros_blk):
    return pl.kernel(
        _deg_kernel,
        out_type=jax.ShapeDtypeStruct((NC, NPAD, DW), jnp.float32),
        mesh=_mesh,
        compiler_params=_SC_PARAMS,
        scratch_types=[
            pltpu.VMEM((CPT, CHUNK), jnp.int32),
            pltpu.VMEM((CHUNK, DW), jnp.float32),
            pltpu.VMEM_SHARED((NPAD, DW), jnp.float32),
            pltpu.SemaphoreType.DMA((NBUF,)),
        ],
    )(ei3, ones_blk, zeros_blk)


def _agg_kernel(hp_hbm, ei_hbm, z_hbm, out_hbm, src_all, dst_all,
                rows, acc, semg, sems):
    cid = lax.axis_index("c")
    sid = lax.axis_index("s")
    wid, nj, cbase, off = _tile_work(cid, sid)
    sl = pl.ds(sid * RPS, RPS)
    pltpu.sync_copy(z_hbm, acc.at[sl])
    pltpu.sync_copy(ei_hbm.at[0, pl.ds(cbase, CPT)], src_all)
    pltpu.sync_copy(ei_hbm.at[1, pl.ds(cbase, CPT)], dst_all)
    plsc.subcore_barrier()

    def g_desc(j, b):
        return pltpu.make_async_copy(
            hp_hbm.at[src_all.at[j + off]], rows.at[b], semg.at[b])

    def s_desc(j, b):
        return pltpu.make_async_copy(
            rows.at[b], acc.at[dst_all.at[j + off]], sems.at[b])

    @pl.loop(0, CPT + DEPTH)
    def _(j):
        @pl.when(j < nj)
        def _():
            b = lax.rem(j, NBUF)

            @pl.when(j >= NBUF)
            def _():
                s_desc(j - NBUF, b).wait()

            g_desc(j, b).start()

        @pl.when(jnp.logical_and(j >= DEPTH, j < nj + DEPTH))
        def _():
            jd = j - DEPTH
            bd = lax.rem(jd, NBUF)
            g_desc(jd, bd).wait()
            s_desc(jd, bd).start(add=True)

    for b in range(NBUF):
        jl = nj - NBUF + b
        s_desc(jl, lax.rem(jl, NBUF)).wait()
    plsc.subcore_barrier()
    pltpu.sync_copy(acc.at[pl.ds(sid * RPS, RPS)],
                    out_hbm.at[cid, pl.ds(sid * RPS, RPS)])


def _agg_call(hp, ei3, zeros_blk):
    return pl.kernel(
        _agg_kernel,
        out_type=jax.ShapeDtypeStruct((NC, NPAD, HID), jnp.float32),
        mesh=_mesh,
        compiler_params=_SC_PARAMS,
        scratch_types=[
            pltpu.VMEM((CPT, CHUNK), jnp.int32),
            pltpu.VMEM((CPT, CHUNK), jnp.int32),
            pltpu.VMEM((NBUF, CHUNK, HID), jnp.float32),
            pltpu.VMEM_SHARED((NPAD, HID), jnp.float32),
            pltpu.SemaphoreType.DMA((NBUF,)),
            pltpu.SemaphoreType.DMA((NBUF,)),
        ],
    )(hp, ei3, zeros_blk)


def _dinv(dp_ref):
    deg = dp_ref[0] + dp_ref[1] + 1.0
    return 1.0 / jnp.sqrt(deg[:, 0:1])


def _k1a_body(x_ref, w_ref, u_ref):
    u_ref[...] = lax.dot_general(
        x_ref[...], w_ref[...], (((1,), (0,)), ((), ())), precision=_P)


def _k1b_body(u_ref, dp_ref, hp_ref):
    hp_ref[...] = u_ref[...] * _dinv(dp_ref)


def _k2_body(p_ref, hp_ref, b_ref, dp_ref, w_ref, o_ref):
    dinv = _dinv(dp_ref)
    a = jnp.maximum(dinv * (p_ref[0] + p_ref[1] + hp_ref[...]) + b_ref[...], 0.0)
    o_ref[...] = lax.dot_general(
        a, w_ref[...], (((1,), (0,)), ((), ())), precision=_P
    ) * dinv


def _k4_body(p_ref, hp_ref, b_ref, dp_ref, batch_ref, wc_ref, bc_ref, o_ref):
    dinv = _dinv(dp_ref)
    a = jnp.maximum(dinv * (p_ref[0] + p_ref[1] + hp_ref[...]) + b_ref[...], 0.0)
    gid = lax.broadcasted_iota(jnp.int32, (NPAD, G), 1)
    oh = (batch_ref[...] == gid).astype(jnp.float32)
    sums = lax.dot_general(oh, a, (((0,), (0,)), ((), ())), precision=_P)
    cnts = lax.dot_general(oh, jnp.ones((NPAD, 1), jnp.float32),
                           (((0,), (0,)), ((), ())), precision=_P)
    pooled = sums / jnp.maximum(cnts, 1.0)
    o_ref[...] = lax.dot_general(
        pooled, wc_ref[...], (((1,), (0,)), ((), ())), precision=_P
    ) + bc_ref[...]


def kernel(x, edge_index, batch, W1, b1, W2, b2, W3, b3, Wc, bc):
    ei3 = edge_index.astype(jnp.int32).reshape(2, NCHUNK, CHUNK)
    xp = jnp.pad(x, ((0, NPAD - N), (0, 0)))
    batchp = jnp.pad(batch.astype(jnp.int32), (0, NPAD - N),
                     constant_values=G).reshape(NPAD, 1)
    z64 = jnp.zeros((RPS, HID), jnp.float32)
    zd = jnp.zeros((RPS, DW), jnp.float32)
    onesd = jnp.ones((CHUNK, DW), jnp.float32)
    b1r, b2r, b3r = b1.reshape(1, HID), b2.reshape(1, HID), b3.reshape(1, HID)
    bcr = bc.reshape(1, TASKS)

    dp = _deg_call(ei3, onesd, zd)

    u1 = pl.pallas_call(
        _k1a_body, out_shape=jax.ShapeDtypeStruct((NPAD, HID), jnp.float32),
    )(xp, W1)
    hp1 = pl.pallas_call(
        _k1b_body, out_shape=jax.ShapeDtypeStruct((NPAD, HID), jnp.float32),
    )(u1, dp)

    p1 = _agg_call(hp1, ei3, z64)
    hp2 = pl.pallas_call(
        _k2_body, out_shape=jax.ShapeDtypeStruct((NPAD, HID), jnp.float32),
    )(p1, hp1, b1r, dp, W2)

    p2 = _agg_call(hp2, ei3, z64)
    hp3 = pl.pallas_call(
        _k2_body, out_shape=jax.ShapeDtypeStruct((NPAD, HID), jnp.float32),
    )(p2, hp2, b2r, dp, W3)

    p3 = _agg_call(hp3, ei3, z64)
    out = pl.pallas_call(
        _k4_body, out_shape=jax.ShapeDtypeStruct((G, TASKS), jnp.float32),
    )(p3, hp3, b3r, dp, batchp, Wc, bcr)
    return out


# speedup vs baseline: 3.7284x; 1.0025x over previous
import jax
import jax.numpy as jnp
from jax import lax
from jax.experimental import pallas as pl
from jax.experimental.pallas import tpu as pltpu
from jax.experimental.pallas import tpu_sc as plsc

N = 10000
E = 320000
IN_CH = 128
HID = 64
G = 64
TASKS = 2

NC, NS = 2, 16
NW = NC * NS
CHUNK = 128
NBUF = 8
DEPTH = 6
NCHUNK = E // CHUNK
CPQ = NCHUNK // NW
CPR = NCHUNK % NW
CPT = CPQ + 1
NPAD = 10240
RPS = NPAD // NS
DW = 16

_P = jax.lax.Precision.HIGHEST
def _mesh():
    return plsc.VectorSubcoreMesh(core_axis_name="c", subcore_axis_name="s",
                                  num_cores=NC, num_subcores=NS)
_SC_PARAMS = pltpu.CompilerParams(use_tc_tiling_on_sc=False)


def _tile_work(cid, sid):
    wid = cid * NS + sid
    nj = jnp.where(wid < CPR, CPQ + 1, CPQ)
    cbase = CPQ * wid + jnp.minimum(wid, CPR)
    off = jnp.where(cbase + CPT > NCHUNK, 1, 0)
    return wid, nj, cbase - off, off


def _deg_kernel(ei_hbm, ones_hbm, z_hbm, out_hbm, dst_all, ones_v, dacc,
                sem):
    cid = lax.axis_index("c")
    sid = lax.axis_index("s")
    wid, nj, cbase, off = _tile_work(cid, sid)
    pltpu.sync_copy(z_hbm, dacc.at[pl.ds(sid * RPS, RPS)])
    pltpu.sync_copy(ones_hbm, ones_v)
    pltpu.sync_copy(ei_hbm.at[1, pl.ds(cbase, CPT)], dst_all)
    plsc.subcore_barrier()

    def s_desc(j, b):
        return pltpu.make_async_copy(
            ones_v, dacc.at[dst_all.at[j + off]], sem.at[b])

    @pl.loop(0, CPT)
    def _(j):
        @pl.when(j < nj)
        def _():
            b = lax.rem(j, NBUF)

            @pl.when(j >= NBUF)
            def _():
                s_desc(j - NBUF, b).wait()

            s_desc(j, b).start(add=True)

    for b in range(NBUF):
        jl = nj - NBUF + b
        s_desc(jl, lax.rem(jl, NBUF)).wait()
    plsc.subcore_barrier()
    pltpu.sync_copy(dacc.at[pl.ds(sid * RPS, RPS)],
                    out_hbm.at[cid, pl.ds(sid * RPS, RPS)])


def _deg_call(ei3, ones_blk, zeros_blk):
    return pl.kernel(
        _deg_kernel,
        out_type=jax.ShapeDtypeStruct((NC, NPAD, DW), jnp.float32),
        mesh=_mesh(),
        compiler_params=_SC_PARAMS,
        scratch_types=[
            pltpu.VMEM((CPT, CHUNK), jnp.int32),
            pltpu.VMEM((CHUNK, DW), jnp.float32),
            pltpu.VMEM_SHARED((NPAD, DW), jnp.float32),
            pltpu.SemaphoreType.DMA((NBUF,)),
        ],
    )(ei3, ones_blk, zeros_blk)


def _agg_kernel(hp_hbm, ei_hbm, z_hbm, out_hbm, src_all, dst_all,
                rows, acc, semg, sems):
    cid = lax.axis_index("c")
    sid = lax.axis_index("s")
    wid, nj, cbase, off = _tile_work(cid, sid)
    sl = pl.ds(sid * RPS, RPS)
    pltpu.sync_copy(z_hbm, acc.at[sl])
    pltpu.sync_copy(ei_hbm.at[0, pl.ds(cbase, CPT)], src_all)
    pltpu.sync_copy(ei_hbm.at[1, pl.ds(cbase, CPT)], dst_all)
    plsc.subcore_barrier()

    def g_desc(j, b):
        return pltpu.make_async_copy(
            hp_hbm.at[src_all.at[j + off]], rows.at[b], semg.at[b])

    def s_desc(j, b):
        return pltpu.make_async_copy(
            rows.at[b], acc.at[dst_all.at[j + off]], sems.at[b])

    @pl.loop(0, CPT + DEPTH)
    def _(j):
        @pl.when(j < nj)
        def _():
            b = lax.rem(j, NBUF)

            @pl.when(j >= NBUF)
            def _():
                s_desc(j - NBUF, b).wait()

            g_desc(j, b).start()

        @pl.when(jnp.logical_and(j >= DEPTH, j < nj + DEPTH))
        def _():
            jd = j - DEPTH
            bd = lax.rem(jd, NBUF)
            g_desc(jd, bd).wait()
            s_desc(jd, bd).start(add=True)

    for b in range(NBUF):
        jl = nj - NBUF + b
        s_desc(jl, lax.rem(jl, NBUF)).wait()
    plsc.subcore_barrier()
    pltpu.sync_copy(acc.at[pl.ds(sid * RPS, RPS)],
                    out_hbm.at[cid, pl.ds(sid * RPS, RPS)])


def _agg_call(hp, ei3, zeros_blk):
    return pl.kernel(
        _agg_kernel,
        out_type=jax.ShapeDtypeStruct((NC, NPAD, HID), jnp.float32),
        mesh=_mesh(),
        compiler_params=_SC_PARAMS,
        scratch_types=[
            pltpu.VMEM((CPT, CHUNK), jnp.int32),
            pltpu.VMEM((CPT, CHUNK), jnp.int32),
            pltpu.VMEM((NBUF, CHUNK, HID), jnp.float32),
            pltpu.VMEM_SHARED((NPAD, HID), jnp.float32),
            pltpu.SemaphoreType.DMA((NBUF,)),
            pltpu.SemaphoreType.DMA((NBUF,)),
        ],
    )(hp, ei3, zeros_blk)


def _dinv(dp_ref):
    deg = dp_ref[0] + dp_ref[1] + 1.0
    return 1.0 / jnp.sqrt(deg[:, 0:1])


def _k1a_body(x_ref, w_ref, u_ref):
    u_ref[...] = lax.dot_general(
        x_ref[...], w_ref[...], (((1,), (0,)), ((), ())), precision=_P)


def _k1b_body(u_ref, dp_ref, hp_ref):
    hp_ref[...] = u_ref[...] * _dinv(dp_ref)


def _k2_body(p_ref, hp_ref, b_ref, dp_ref, w_ref, o_ref):
    dinv = _dinv(dp_ref)
    a = jnp.maximum(dinv * (p_ref[0] + p_ref[1] + hp_ref[...]) + b_ref[...], 0.0)
    o_ref[...] = lax.dot_general(
        a, w_ref[...], (((1,), (0,)), ((), ())), precision=_P
    ) * dinv


def _k4_body(p_ref, hp_ref, b_ref, dp_ref, batch_ref, wc_ref, bc_ref, o_ref):
    dinv = _dinv(dp_ref)
    a = jnp.maximum(dinv * (p_ref[0] + p_ref[1] + hp_ref[...]) + b_ref[...], 0.0)
    gid = lax.broadcasted_iota(jnp.int32, (NPAD, G), 1)
    oh = (batch_ref[...] == gid).astype(jnp.float32)
    sums = lax.dot_general(oh, a, (((0,), (0,)), ((), ())), precision=_P)
    cnts = lax.dot_general(oh, jnp.ones((NPAD, 1), jnp.float32),
                           (((0,), (0,)), ((), ())), precision=_P)
    pooled = sums / jnp.maximum(cnts, 1.0)
    o_ref[...] = lax.dot_general(
        pooled, wc_ref[...], (((1,), (0,)), ((), ())), precision=_P
    ) + bc_ref[...]


def kernel(x, edge_index, batch, W1, b1, W2, b2, W3, b3, Wc, bc):
    ei3 = edge_index.astype(jnp.int32).reshape(2, NCHUNK, CHUNK)
    xp = jnp.pad(x, ((0, NPAD - N), (0, 0)))
    batchp = jnp.pad(batch.astype(jnp.int32), (0, NPAD - N),
                     constant_values=G).reshape(NPAD, 1)
    z64 = jnp.zeros((RPS, HID), jnp.float32)
    zd = jnp.zeros((RPS, DW), jnp.float32)
    onesd = jnp.ones((CHUNK, DW), jnp.float32)
    b1r, b2r, b3r = b1.reshape(1, HID), b2.reshape(1, HID), b3.reshape(1, HID)
    bcr = bc.reshape(1, TASKS)

    dp = _deg_call(ei3, onesd, zd)

    u1 = pl.pallas_call(
        _k1a_body, out_shape=jax.ShapeDtypeStruct((NPAD, HID), jnp.float32),
    )(xp, W1)
    hp1 = pl.pallas_call(
        _k1b_body, out_shape=jax.ShapeDtypeStruct((NPAD, HID), jnp.float32),
    )(u1, dp)

    p1 = _agg_call(hp1, ei3, z64)
    hp2 = pl.pallas_call(
        _k2_body, out_shape=jax.ShapeDtypeStruct((NPAD, HID), jnp.float32),
    )(p1, hp1, b1r, dp, W2)

    p2 = _agg_call(hp2, ei3, z64)
    hp3 = pl.pallas_call(
        _k2_body, out_shape=jax.ShapeDtypeStruct((NPAD, HID), jnp.float32),
    )(p2, hp2, b2r, dp, W3)

    p3 = _agg_call(hp3, ei3, z64)
    out = pl.pallas_call(
        _k4_body, out_shape=jax.ShapeDtypeStruct((G, TASKS), jnp.float32),
    )(p3, hp3, b3r, dp, batchp, Wc, bcr)
    return out
